# baseline (speedup 1.0000x reference)
import functools

import jax
import jax.numpy as jnp
from jax import lax
from jax.experimental import pallas as pl
from jax.experimental.pallas import tpu as pltpu
from jax.experimental.pallas import tpu_sc as plsc

_NC = 2
_NS = 16
_NW = _NC * _NS

_CHUNK = 32768


def _sc_body(n_chunks, in_hbm, out_hbm, buf, rsem, wsem):
    wid = lax.axis_index("s") * _NC + lax.axis_index("c")
    base = wid * (n_chunks * _CHUNK)

    def read(g, slot):
        return pltpu.async_copy(
            in_hbm.at[pl.ds(base + g * _CHUNK, _CHUNK)], buf.at[slot], rsem)

    def write(g, slot):
        return pltpu.async_copy(
            buf.at[slot], out_hbm.at[pl.ds(base + g * _CHUNK, _CHUNK)], wsem)

    read(0, 0)
    for g in range(n_chunks):
        cur, nxt = g % 2, (g + 1) % 2
        pltpu.make_async_copy(
            in_hbm.at[pl.ds(base + g * _CHUNK, _CHUNK)], buf.at[cur], rsem
        ).wait()
        if g + 1 < n_chunks:
            read(g + 1, nxt)
        w = write(g, cur)
        w.wait()


def kernel(phi):
    shape = phi.shape
    flat = phi.reshape(-1)
    n = flat.shape[0]
    assert n % (_NW * _CHUNK) == 0
    n_chunks = n // (_NW * _CHUNK)

    mesh = plsc.VectorSubcoreMesh(core_axis_name="c", subcore_axis_name="s")
    run = pl.kernel(
        functools.partial(_sc_body, n_chunks),
        mesh=mesh,
        out_type=jax.ShapeDtypeStruct((n,), flat.dtype),
        scratch_types=[
            pltpu.VMEM((2, _CHUNK), jnp.float32),
            pltpu.SemaphoreType.DMA,
            pltpu.SemaphoreType.DMA,
        ],
    )
    return run(flat).reshape(shape)


# speedup vs baseline: 1.0027x; 1.0027x over previous
import functools

import jax
import jax.numpy as jnp
from jax import lax
from jax.experimental import pallas as pl
from jax.experimental.pallas import tpu as pltpu
from jax.experimental.pallas import tpu_sc as plsc

_NC = 2
_NS = 16
_NW = _NC * _NS

_CHUNK = 32768


_NBUF = 3
_RA = 2


def _sc_body(n_chunks, in_hbm, out_hbm, buf, rsem, wsem):
    wid = lax.axis_index("s") * _NC + lax.axis_index("c")
    base = wid * (n_chunks * _CHUNK)

    def read(g, slot):
        return pltpu.async_copy(
            in_hbm.at[pl.ds(base + g * _CHUNK, _CHUNK)], buf.at[slot], rsem)

    def wait_read(g, slot):
        pltpu.make_async_copy(
            in_hbm.at[pl.ds(base + g * _CHUNK, _CHUNK)], buf.at[slot], rsem
        ).wait()

    def write(g, slot):
        return pltpu.async_copy(
            buf.at[slot], out_hbm.at[pl.ds(base + g * _CHUNK, _CHUNK)], wsem)

    def wait_write(g, slot):
        pltpu.make_async_copy(
            buf.at[slot], out_hbm.at[pl.ds(base + g * _CHUNK, _CHUNK)], wsem
        ).wait()

    ra = min(_RA, n_chunks)
    for g in range(ra):
        read(g, g % _NBUF)
    waited = -1
    for g in range(n_chunks):
        cur = g % _NBUF
        wait_read(g, cur)
        write(g, cur)
        nxt = g + ra
        if nxt < n_chunks:
            conflict = nxt - _NBUF
            while waited < conflict:
                waited += 1
                wait_write(waited, waited % _NBUF)
            read(nxt, nxt % _NBUF)
    while waited < n_chunks - 1:
        waited += 1
        wait_write(waited, waited % _NBUF)


def kernel(phi):
    shape = phi.shape
    flat = phi.reshape(-1)
    n = flat.shape[0]
    assert n % (_NW * _CHUNK) == 0
    n_chunks = n // (_NW * _CHUNK)

    mesh = plsc.VectorSubcoreMesh(core_axis_name="c", subcore_axis_name="s")
    run = pl.kernel(
        functools.partial(_sc_body, n_chunks),
        mesh=mesh,
        out_type=jax.ShapeDtypeStruct((n,), flat.dtype),
        scratch_types=[
            pltpu.VMEM((2, _CHUNK), jnp.float32),
            pltpu.SemaphoreType.DMA,
            pltpu.SemaphoreType.DMA,
        ],
    )
    return run(flat).reshape(shape)
